# baseline (speedup 1.0000x reference)
import functools

import jax
import jax.numpy as jnp
from jax import lax
from jax.experimental import pallas as pl
from jax.experimental.pallas import tpu as pltpu
from jax.experimental.pallas import tpu_sc as plsc

B = 16
P = 1024
KNN = 32
K1 = KNN + 1
N = B * P
M = B * K1 * P
EPS = 1e-5
_R = 0.2
_RR = _R * _R
PREC = lax.Precision.HIGHEST


def _dot(a, b):
    return jnp.dot(a, b, preferred_element_type=jnp.float32, precision=PREC)


def _nbr_body(posp_ref, post_ref, xp_ref, w1_ref, wp1_ref, b1_ref,
              nbrg_ref, valid_ref, s1_ref, tb1_ref):
    b = pl.program_id(0)
    pp = posp_ref[0]
    lane = lax.broadcasted_iota(jnp.int32, (P, P), 1)
    sub = lax.broadcasted_iota(jnp.int32, (P, P), 0)
    key = jnp.zeros((P, P), jnp.float32)
    for c in range(3):
        d = pp[:, c:c + 1] - post_ref[0, c:c + 1, :]
        key = key + d * d
    key = jnp.where((key < _RR) & (lane != sub), key, jnp.inf)
    selfidx = lax.broadcasted_iota(jnp.int32, (P,), 0)
    for t in range(KNN):
        m = jnp.min(key, axis=1)
        idx = jnp.argmin(key, axis=1).astype(jnp.int32)
        ok = m < _RR
        nbr_t = jnp.where(ok, idx, selfidx)
        nbrg_ref[0, t, 0, :] = nbr_t + b * P
        valid_ref[0, t, 0, :] = ok.astype(jnp.float32)
        key = jnp.where(lane == idx[:, None], jnp.inf, key)
    nbrg_ref[0, KNN, 0, :] = selfidx + b * P
    valid_ref[0, KNN, 0, :] = jnp.ones((P,), jnp.float32)
    s1_ref[0] = _dot(xp_ref[0], w1_ref[...])
    tb1_ref[0] = b1_ref[...] - _dot(pp, wp1_ref[...])


def _build_graph(posp, post, xp, w1p, wp1, b1):
    big = lambda b: (b, 0, 0)
    big4 = lambda b: (b, 0, 0, 0)
    fix = lambda b: (0, 0)
    return pl.pallas_call(
        _nbr_body,
        grid=(B,),
        in_specs=[
            pl.BlockSpec((1, P, 8), big),
            pl.BlockSpec((1, 8, P), big),
            pl.BlockSpec((1, P, 8), big),
            pl.BlockSpec((8, 64), fix),
            pl.BlockSpec((8, 64), fix),
            pl.BlockSpec((1, 64), fix),
        ],
        out_specs=[
            pl.BlockSpec((1, K1, 1, P), big4),
            pl.BlockSpec((1, K1, 1, P), big4),
            pl.BlockSpec((1, P, 64), big),
            pl.BlockSpec((1, P, 64), big),
        ],
        out_shape=[
            jax.ShapeDtypeStruct((B, K1, 1, P), jnp.int32),
            jax.ShapeDtypeStruct((B, K1, 1, P), jnp.float32),
            jax.ShapeDtypeStruct((B, P, 64), jnp.float32),
            jax.ShapeDtypeStruct((B, P, 64), jnp.float32),
        ],
    )(posp, post, xp, w1p, wp1, b1)


def _accum_stats(st_ref, r, rm, vsum, first):
    @pl.when(first)
    def _():
        st_ref[...] = jnp.zeros_like(st_ref)

    C = r.shape[1]
    st_ref[0:1, :] = st_ref[0:1, :] + jnp.sum(rm, 0, keepdims=True)
    st_ref[1:2, :] = st_ref[1:2, :] + jnp.sum(rm * r, 0, keepdims=True)
    st_ref[2:3, :] = st_ref[2:3, :] + jnp.full((1, C), vsum)


def _finalize(st_ref):
    cnt = jnp.maximum(st_ref[2, 0], 1.0)
    mu = st_ref[0:1, :] / cnt
    var = st_ref[1:2, :] / cnt - mu * mu
    return mu, lax.rsqrt(var + EPS)


_NW = 32
_CH = 128


def _sc_gather(table, idx):
    Mr = idx.shape[0]
    C = table.shape[1]
    per = Mr // _NW
    nch = per // _CH
    mesh = plsc.VectorSubcoreMesh(core_axis_name="c", subcore_axis_name="s")

    @functools.partial(
        pl.kernel, mesh=mesh,
        out_type=jax.ShapeDtypeStruct((Mr, C), jnp.float32),
        scratch_types=[
            pltpu.VMEM((_CH,), jnp.int32),
            pltpu.VMEM((_CH,), jnp.int32),
            pltpu.VMEM((_CH, C), jnp.float32),
            pltpu.VMEM((_CH, C), jnp.float32),
            pltpu.SemaphoreType.DMA,
            pltpu.SemaphoreType.DMA,
            pltpu.SemaphoreType.DMA,
            pltpu.SemaphoreType.DMA,
        ],
    )
    def k(table_hbm, idx_hbm, out_hbm, i0, i1, r0, r1, sg0, sg1, sw0, sw1):
        wid = lax.axis_index("s") * 2 + lax.axis_index("c")
        base = wid * per
        iv = (i0, i1)
        rv = (r0, r1)
        sg = (sg0, sg1)
        sw = (sw0, sw1)

        def wait_gather(b):
            pltpu.make_async_copy(table_hbm.at[pl.ds(0, _CH)], rv[b],
                                  sg[b]).wait()

        def wait_wb(b, off):
            pltpu.make_async_copy(rv[b], out_hbm.at[pl.ds(off, _CH)],
                                  sw[b]).wait()

        for b in range(2):
            pltpu.sync_copy(idx_hbm.at[pl.ds(base + b * _CH, _CH)], iv[b])
            pltpu.async_copy(table_hbm.at[iv[b]], rv[b], sg[b])

        def body(j2, carry):
            for b in range(2):
                off = base + (j2 * 2 + b) * _CH
                wait_gather(b)
                pltpu.async_copy(rv[b], out_hbm.at[pl.ds(off, _CH)], sw[b])
                pltpu.sync_copy(idx_hbm.at[pl.ds(off + 2 * _CH, _CH)], iv[b])
                wait_wb(b, off)
                pltpu.async_copy(table_hbm.at[iv[b]], rv[b], sg[b])
            return carry

        lax.fori_loop(0, nch // 2 - 1, body, 0)
        for b in range(2):
            off = base + (nch - 2 + b) * _CH
            wait_gather(b)
            pltpu.sync_copy(rv[b], out_hbm.at[pl.ds(off, _CH)])

    return k(table, idx)


_KB = 3
_KL = K1 // _KB


def _gstat_body(g_ref, valid_ref, tb_ref, st_ref, *, ct):
    b, kb = pl.program_id(0), pl.program_id(1)
    tb = tb_ref[0]
    s0 = jnp.zeros((1, ct), jnp.float32)
    s1 = jnp.zeros((1, ct), jnp.float32)
    vs = 0.0
    for kk in range(_KL):
        r = jnp.maximum(g_ref[0, kk][:, :ct] + tb, 0.0)
        v = valid_ref[0, kk, 0, :]
        rm = r * v[:, None]
        s0 = s0 + jnp.sum(rm, 0, keepdims=True)
        s1 = s1 + jnp.sum(rm * r, 0, keepdims=True)
        vs = vs + jnp.sum(v)

    @pl.when((b == 0) & (kb == 0))
    def _():
        st_ref[...] = jnp.zeros_like(st_ref)

    st_ref[0:1, :] = st_ref[0:1, :] + s0
    st_ref[1:2, :] = st_ref[1:2, :] + s1
    st_ref[2:3, :] = st_ref[2:3, :] + jnp.full((1, ct), vs)


def _gather_stats(g4, valid, tb):
    Cg = g4.shape[-1]
    ct = tb.shape[-1]
    return pl.pallas_call(
        functools.partial(_gstat_body, ct=ct),
        grid=(B, _KB),
        in_specs=[
            pl.BlockSpec((1, _KL, P, Cg), lambda b, k: (b, k, 0, 0)),
            pl.BlockSpec((1, _KL, 1, P), lambda b, k: (b, k, 0, 0)),
            pl.BlockSpec((1, P, ct), lambda b, k: (b, 0, 0)),
        ],
        out_specs=pl.BlockSpec((8, ct), lambda b, k: (0, 0)),
        out_shape=jax.ShapeDtypeStruct((8, ct), jnp.float32),
    )(g4, valid, tb)


def _midg_body(g_ref, valid_ref, tb_ref, sti_ref, w_ref, b_ref,
               r_ref, st_ref, *, ct):
    b, kb = pl.program_id(0), pl.program_id(1)
    tb = tb_ref[0]
    mu, inv = _finalize(sti_ref)
    w = w_ref[...]
    bv = b_ref[...]
    Cout = w.shape[1]
    s0 = jnp.zeros((1, Cout), jnp.float32)
    s1 = jnp.zeros((1, Cout), jnp.float32)
    vs = 0.0
    for kk in range(_KL):
        r1 = jnp.maximum(g_ref[0, kk][:, :ct] + tb, 0.0)
        h = (r1 - mu) * inv
        r = jnp.maximum(_dot(h, w) + bv, 0.0)
        r_ref[0, kk] = r
        v = valid_ref[0, kk, 0, :]
        rm = r * v[:, None]
        s0 = s0 + jnp.sum(rm, 0, keepdims=True)
        s1 = s1 + jnp.sum(rm * r, 0, keepdims=True)
        vs = vs + jnp.sum(v)

    @pl.when((b == 0) & (kb == 0))
    def _():
        st_ref[...] = jnp.zeros_like(st_ref)

    st_ref[0:1, :] = st_ref[0:1, :] + s0
    st_ref[1:2, :] = st_ref[1:2, :] + s1
    st_ref[2:3, :] = st_ref[2:3, :] + jnp.full((1, Cout), vs)


def _mid_gathered(g4, valid, tb, sti, w, bvec):
    Cg = g4.shape[-1]
    ct = tb.shape[-1]
    Cout = w.shape[1]
    return pl.pallas_call(
        functools.partial(_midg_body, ct=ct),
        grid=(B, _KB),
        in_specs=[
            pl.BlockSpec((1, _KL, P, Cg), lambda b, k: (b, k, 0, 0)),
            pl.BlockSpec((1, _KL, 1, P), lambda b, k: (b, k, 0, 0)),
            pl.BlockSpec((1, P, ct), lambda b, k: (b, 0, 0)),
            pl.BlockSpec((8, ct), lambda b, k: (0, 0)),
            pl.BlockSpec((ct, Cout), lambda b, k: (0, 0)),
            pl.BlockSpec((1, Cout), lambda b, k: (0, 0)),
        ],
        out_specs=[
            pl.BlockSpec((1, _KL, P, Cout), lambda b, k: (b, k, 0, 0)),
            pl.BlockSpec((8, Cout), lambda b, k: (0, 0)),
        ],
        out_shape=[
            jax.ShapeDtypeStruct((B, K1, P, Cout), jnp.float32),
            jax.ShapeDtypeStruct((8, Cout), jnp.float32),
        ],
    )(g4, valid, tb, sti, w, bvec)


def _mid_body(a_ref, v_ref, sti_ref, w_ref, b_ref, r_ref, st_ref):
    i = pl.program_id(0)
    mu, inv = _finalize(sti_ref)
    h = (a_ref[...] - mu) * inv
    z = _dot(h, w_ref[...]) + b_ref[...]
    r = jnp.maximum(z, 0.0)
    r_ref[...] = r
    v = v_ref[...]
    _accum_stats(st_ref, r, r * v, jnp.sum(v), i == 0)


def _mid(a, v, sti, w, bvec, bm):
    Mr, Cin = a.shape
    Cout = w.shape[1]
    return pl.pallas_call(
        _mid_body,
        grid=(Mr // bm,),
        in_specs=[
            pl.BlockSpec((bm, Cin), lambda i: (i, 0)),
            pl.BlockSpec((bm, 1), lambda i: (i, 0)),
            pl.BlockSpec((8, Cin), lambda i: (0, 0)),
            pl.BlockSpec((Cin, Cout), lambda i: (0, 0)),
            pl.BlockSpec((1, Cout), lambda i: (0, 0)),
        ],
        out_specs=[
            pl.BlockSpec((bm, Cout), lambda i: (i, 0)),
            pl.BlockSpec((8, Cout), lambda i: (0, 0)),
        ],
        out_shape=[
            jax.ShapeDtypeStruct((Mr, Cout), jnp.float32),
            jax.ShapeDtypeStruct((8, Cout), jnp.float32),
        ],
    )(a, v, sti, w, bvec)


def _mid2_body(a_ref, b2_ref, v_ref, sti_ref, wa_ref, wb_ref, b_ref,
               r_ref, st_ref):
    i = pl.program_id(0)
    mu, inv = _finalize(sti_ref)
    h = (a_ref[...] - mu) * inv
    z = _dot(h, wa_ref[...]) + _dot(b2_ref[...], wb_ref[...]) + b_ref[...]
    r = jnp.maximum(z, 0.0)
    r_ref[...] = r
    v = v_ref[...]
    _accum_stats(st_ref, r, r * v, jnp.sum(v), i == 0)


def _mid2(a, b2, v, sti, wa, wb, bvec, bm):
    Mr, Cin = a.shape
    Cb = b2.shape[1]
    Cout = wa.shape[1]
    return pl.pallas_call(
        _mid2_body,
        grid=(Mr // bm,),
        in_specs=[
            pl.BlockSpec((bm, Cin), lambda i: (i, 0)),
            pl.BlockSpec((bm, Cb), lambda i: (i, 0)),
            pl.BlockSpec((bm, 1), lambda i: (i, 0)),
            pl.BlockSpec((8, Cin), lambda i: (0, 0)),
            pl.BlockSpec((Cin, Cout), lambda i: (0, 0)),
            pl.BlockSpec((Cb, Cout), lambda i: (0, 0)),
            pl.BlockSpec((1, Cout), lambda i: (0, 0)),
        ],
        out_specs=[
            pl.BlockSpec((bm, Cout), lambda i: (i, 0)),
            pl.BlockSpec((8, Cout), lambda i: (0, 0)),
        ],
        out_shape=[
            jax.ShapeDtypeStruct((Mr, Cout), jnp.float32),
            jax.ShapeDtypeStruct((8, Cout), jnp.float32),
        ],
    )(a, b2, v, sti, wa, wb, bvec)


def _last_body(a_ref, valid_ref, sti_ref, w_ref, b_ref, pool_ref, st_ref):
    b, kb = pl.program_id(0), pl.program_id(1)
    mu, inv = _finalize(sti_ref)
    w = w_ref[...]
    bv = b_ref[...]
    Cout = w.shape[1]
    s0 = jnp.zeros((1, Cout), jnp.float32)
    s1 = jnp.zeros((1, Cout), jnp.float32)
    vs = 0.0

    @pl.when(kb == 0)
    def _():
        pool_ref[0] = jnp.full_like(pool_ref[0], -jnp.inf)

    acc = pool_ref[0]
    for kk in range(_KL):
        h = (a_ref[0, kk] - mu) * inv
        r = jnp.maximum(_dot(h, w) + bv, 0.0)
        v = valid_ref[0, kk, 0, :]
        rm = r * v[:, None]
        s0 = s0 + jnp.sum(rm, 0, keepdims=True)
        s1 = s1 + jnp.sum(rm * r, 0, keepdims=True)
        vs = vs + jnp.sum(v)
        acc = jnp.maximum(acc, jnp.where(v[:, None] > 0, r, -jnp.inf))
    pool_ref[0] = acc

    @pl.when((b == 0) & (kb == 0))
    def _():
        st_ref[...] = jnp.zeros_like(st_ref)

    st_ref[0:1, :] = st_ref[0:1, :] + s0
    st_ref[1:2, :] = st_ref[1:2, :] + s1
    st_ref[2:3, :] = st_ref[2:3, :] + jnp.full((1, Cout), vs)


def _conv_last(a4, valid, sti, w, bvec):
    Cin = a4.shape[-1]
    Cout = w.shape[1]
    return pl.pallas_call(
        _last_body,
        grid=(B, _KB),
        in_specs=[
            pl.BlockSpec((1, _KL, P, Cin), lambda b, k: (b, k, 0, 0)),
            pl.BlockSpec((1, _KL, 1, P), lambda b, k: (b, k, 0, 0)),
            pl.BlockSpec((8, Cin), lambda b, k: (0, 0)),
            pl.BlockSpec((Cin, Cout), lambda b, k: (0, 0)),
            pl.BlockSpec((1, Cout), lambda b, k: (0, 0)),
        ],
        out_specs=[
            pl.BlockSpec((1, P, Cout), lambda b, k: (b, 0, 0)),
            pl.BlockSpec((8, Cout), lambda b, k: (0, 0)),
        ],
        out_shape=[
            jax.ShapeDtypeStruct((B, P, Cout), jnp.float32),
            jax.ShapeDtypeStruct((8, Cout), jnp.float32),
        ],
    )(a4, valid, sti, w, bvec)


def _p4_body(a_ref, sti_ref, pp_ref, wx_ref, wp_ref, b_ref, s2_ref, tb2_ref):
    mu, inv = _finalize(sti_ref)
    h = (a_ref[...] - mu) * inv
    ppw = _dot(pp_ref[...], wp_ref[...])
    s2_ref[...] = _dot(h, wx_ref[...]) + ppw
    tb2_ref[...] = b_ref[...] - ppw


def _point_pre(a, sti, pp, wx, wp, bvec, bm):
    Mr, Cin = a.shape
    Cout = wx.shape[1]
    return pl.pallas_call(
        _p4_body,
        grid=(Mr // bm,),
        in_specs=[
            pl.BlockSpec((bm, Cin), lambda i: (i, 0)),
            pl.BlockSpec((8, Cin), lambda i: (0, 0)),
            pl.BlockSpec((bm, 8), lambda i: (i, 0)),
            pl.BlockSpec((Cin, Cout), lambda i: (0, 0)),
            pl.BlockSpec((8, Cout), lambda i: (0, 0)),
            pl.BlockSpec((1, Cout), lambda i: (0, 0)),
        ],
        out_specs=[
            pl.BlockSpec((bm, Cout), lambda i: (i, 0)),
            pl.BlockSpec((bm, Cout), lambda i: (i, 0)),
        ],
        out_shape=[
            jax.ShapeDtypeStruct((Mr, Cout), jnp.float32),
            jax.ShapeDtypeStruct((Mr, Cout), jnp.float32),
        ],
    )(a, sti, pp, wx, wp, bvec)


def _plast_body(a_ref, sti_ref, w_ref, b_ref, g_ref, st_ref):
    b = pl.program_id(0)
    mu, inv = _finalize(sti_ref)
    h = (a_ref[0] - mu) * inv
    z = _dot(h, w_ref[...]) + b_ref[...]
    r = jnp.maximum(z, 0.0)
    _accum_stats(st_ref, r, r, float(P), b == 0)
    g_ref[0, 0, :] = jnp.max(r, axis=0)


def _pool_last(a3, sti, w, bvec):
    Cin = a3.shape[-1]
    Cout = w.shape[1]
    return pl.pallas_call(
        _plast_body,
        grid=(B,),
        in_specs=[
            pl.BlockSpec((1, P, Cin), lambda b: (b, 0, 0)),
            pl.BlockSpec((8, Cin), lambda b: (0, 0)),
            pl.BlockSpec((Cin, Cout), lambda b: (0, 0)),
            pl.BlockSpec((1, Cout), lambda b: (0, 0)),
        ],
        out_specs=[
            pl.BlockSpec((1, 1, Cout), lambda b: (b, 0, 0)),
            pl.BlockSpec((8, Cout), lambda b: (0, 0)),
        ],
        out_shape=[
            jax.ShapeDtypeStruct((B, 1, Cout), jnp.float32),
            jax.ShapeDtypeStruct((8, Cout), jnp.float32),
        ],
    )(a3, sti, w, bvec)


def _head_body(g_ref, stp_ref, w1_ref, b1_ref, w2_ref, b2_ref, w3_ref,
               b3_ref, out_ref):
    mu, inv = _finalize(stp_ref)
    g = (g_ref[...] - mu) * inv
    h = jnp.maximum(_dot(g, w1_ref[...]) + b1_ref[...], 0.0)
    m1 = jnp.mean(h, 0, keepdims=True)
    v1 = jnp.mean((h - m1) ** 2, 0, keepdims=True)
    h = (h - m1) * lax.rsqrt(v1 + EPS)
    h = jnp.maximum(_dot(h, w2_ref[...]) + b2_ref[...], 0.0)
    m2 = jnp.mean(h, 0, keepdims=True)
    v2 = jnp.mean((h - m2) ** 2, 0, keepdims=True)
    h = (h - m2) * lax.rsqrt(v2 + EPS)
    out_ref[...] = _dot(h, w3_ref[...]) + b3_ref[...]


def _head(g, stp, w1, b1, w2, b2, w3p, b3p):
    return pl.pallas_call(
        _head_body,
        out_shape=jax.ShapeDtypeStruct((B, 128), jnp.float32),
    )(g, stp, w1, b1, w2, b2, w3p, b3p)


def _padrows(w, rows):
    return jnp.concatenate(
        [w, jnp.zeros((rows - w.shape[0], w.shape[1]), jnp.float32)], 0)


def kernel(x, pos, batch, params):
    f32 = jnp.float32
    x = x.astype(f32)
    pos = pos.astype(f32)
    xp = jnp.concatenate([x, pos, jnp.zeros((N, 2), f32)], 1).reshape(B, P, 8)
    posp = jnp.concatenate([pos, jnp.zeros((N, 5), f32)], 1).reshape(B, P, 8)
    post = jnp.swapaxes(posp, 1, 2)

    c1 = params["conv1"]
    w1p = _padrows(c1[0]["W"], 8)
    wp1 = _padrows(c1[0]["W"][3:6], 8)
    b1 = c1[0]["b"][None]

    nbrg, valid, s1, tb1 = _build_graph(posp, post, xp, w1p, wp1, b1)
    bmN = 2048 if N % 2048 == 0 else P

    idx = nbrg.reshape(M)
    s1p = jnp.concatenate([s1.reshape(N, 64), jnp.zeros((N, 64), f32)], 1)
    g1 = _sc_gather(s1p, idx).reshape(B, K1, P, 128)
    st1 = _gather_stats(g1, valid, tb1)
    r2, st2 = _mid_gathered(g1, valid, tb1, st1, c1[1]["W"], c1[1]["b"][None])
    pooled1, st3 = _conv_last(r2, valid, st2, c1[2]["W"], c1[2]["b"][None])

    c2 = params["conv2"]
    wx2 = c2[0]["W"][:128]
    wp2 = _padrows(c2[0]["W"][128:131], 8)
    s2, tb2 = _point_pre(pooled1.reshape(N, 128), st3, posp.reshape(N, 8),
                         wx2, wp2, c2[0]["b"][None], bm=bmN)

    tb2b = tb2.reshape(B, P, 128)
    g2 = _sc_gather(s2, idx).reshape(B, K1, P, 128)
    st1c = _gather_stats(g2, valid, tb2b)
    r2c, st2c = _mid_gathered(g2, valid, tb2b, st1c, c2[1]["W"],
                              c2[1]["b"][None])
    pooled2, st3c = _conv_last(r2c, valid, st2c, c2[2]["W"], c2[2]["b"][None])

    pw = params["pool"]
    wa = pw[0]["W"][:256]
    wb = _padrows(pw[0]["W"][256:259], 8)
    ones = jnp.ones((N, 1), f32)
    rp1, stp1 = _mid2(pooled2.reshape(N, 256), posp.reshape(N, 8), ones,
                      st3c, wa, wb, pw[0]["b"][None], bm=bmN)
    rp2, stp2 = _mid(rp1, ones, stp1, pw[1]["W"], pw[1]["b"][None], bm=bmN)
    g_pre, stp3 = _pool_last(rp2.reshape(B, P, 512), stp2, pw[2]["W"],
                             pw[2]["b"][None])

    fc1, fc2, fc3 = params["fc1"], params["fc2"], params["fc3"]
    w3p = jnp.concatenate([fc3["W"], jnp.zeros((256, 127), f32)], 1)
    b3p = jnp.concatenate([fc3["b"], jnp.zeros((127,), f32)])[None]
    out = _head(g_pre.reshape(B, 1024), stp3, fc1["W"], fc1["b"][None],
                fc2["W"], fc2["b"][None], w3p, b3p)
    return out[:, :1]


# speedup vs baseline: 1.3717x; 1.3717x over previous
import functools

import jax
import jax.numpy as jnp
from jax import lax
from jax.experimental import pallas as pl
from jax.experimental.pallas import tpu as pltpu
from jax.experimental.pallas import tpu_sc as plsc

B = 16
P = 1024
KNN = 32
K1 = KNN + 1
N = B * P
M = B * K1 * P
EPS = 1e-5
_R = 0.2
_RR = _R * _R
PREC = lax.Precision.HIGHEST


def _dot(a, b):
    return jnp.dot(a, b, preferred_element_type=jnp.float32, precision=PREC)


def _nbr_body(posp_ref, post_ref, xp_ref, w1_ref, wp1_ref, b1_ref,
              nbrg_ref, valid_ref, s1_ref, tb1_ref):
    b = pl.program_id(0)
    pp = posp_ref[0]
    lane = lax.broadcasted_iota(jnp.int32, (P, P), 1)
    sub = lax.broadcasted_iota(jnp.int32, (P, P), 0)
    key = jnp.zeros((P, P), jnp.float32)
    for c in range(3):
        d = pp[:, c:c + 1] - post_ref[0, c:c + 1, :]
        key = key + d * d
    key = jnp.where((key < _RR) & (lane != sub), key, jnp.inf)
    selfidx = lax.broadcasted_iota(jnp.int32, (P,), 0)
    for t in range(KNN):
        m = jnp.min(key, axis=0)
        sel = key == m[None, :]
        idx = jnp.min(jnp.where(sel, sub, P), axis=0)
        ok = m < _RR
        nbr_t = jnp.where(ok, idx, selfidx)
        nbrg_ref[0, t, 0, :] = nbr_t + b * P
        valid_ref[0, t, 0, :] = ok.astype(jnp.float32)
        key = jnp.where(sub == idx[None, :], jnp.inf, key)
    nbrg_ref[0, KNN, 0, :] = selfidx + b * P
    valid_ref[0, KNN, 0, :] = jnp.ones((P,), jnp.float32)
    s1_ref[0] = _dot(xp_ref[0], w1_ref[...])
    tb1_ref[0] = b1_ref[...] - _dot(pp, wp1_ref[...])


def _build_graph(posp, post, xp, w1p, wp1, b1):
    big = lambda b: (b, 0, 0)
    big4 = lambda b: (b, 0, 0, 0)
    fix = lambda b: (0, 0)
    return pl.pallas_call(
        _nbr_body,
        grid=(B,),
        in_specs=[
            pl.BlockSpec((1, P, 8), big),
            pl.BlockSpec((1, 8, P), big),
            pl.BlockSpec((1, P, 8), big),
            pl.BlockSpec((8, 64), fix),
            pl.BlockSpec((8, 64), fix),
            pl.BlockSpec((1, 64), fix),
        ],
        out_specs=[
            pl.BlockSpec((1, K1, 1, P), big4),
            pl.BlockSpec((1, K1, 1, P), big4),
            pl.BlockSpec((1, P, 64), big),
            pl.BlockSpec((1, P, 64), big),
        ],
        out_shape=[
            jax.ShapeDtypeStruct((B, K1, 1, P), jnp.int32),
            jax.ShapeDtypeStruct((B, K1, 1, P), jnp.float32),
            jax.ShapeDtypeStruct((B, P, 64), jnp.float32),
            jax.ShapeDtypeStruct((B, P, 64), jnp.float32),
        ],
    )(posp, post, xp, w1p, wp1, b1)


def _accum_stats(st_ref, r, rm, vsum, first):
    @pl.when(first)
    def _():
        st_ref[...] = jnp.zeros_like(st_ref)

    C = r.shape[1]
    st_ref[0:1, :] = st_ref[0:1, :] + jnp.sum(rm, 0, keepdims=True)
    st_ref[1:2, :] = st_ref[1:2, :] + jnp.sum(rm * r, 0, keepdims=True)
    st_ref[2:3, :] = st_ref[2:3, :] + jnp.full((1, C), vsum)


def _finalize(st_ref):
    cnt = jnp.maximum(st_ref[2, 0], 1.0)
    mu = st_ref[0:1, :] / cnt
    var = st_ref[1:2, :] / cnt - mu * mu
    return mu, lax.rsqrt(var + EPS)


_NW = 32
_CH = 128


def _sc_gather(table, idx):
    Mr = idx.shape[0]
    C = table.shape[1]
    per = Mr // _NW
    nch = per // _CH
    mesh = plsc.VectorSubcoreMesh(core_axis_name="c", subcore_axis_name="s")

    @functools.partial(
        pl.kernel, mesh=mesh,
        out_type=jax.ShapeDtypeStruct((Mr, C), jnp.float32),
        scratch_types=[
            pltpu.VMEM((_CH,), jnp.int32),
            pltpu.VMEM((_CH,), jnp.int32),
            pltpu.VMEM((_CH, C), jnp.float32),
            pltpu.VMEM((_CH, C), jnp.float32),
            pltpu.SemaphoreType.DMA,
            pltpu.SemaphoreType.DMA,
            pltpu.SemaphoreType.DMA,
            pltpu.SemaphoreType.DMA,
        ],
    )
    def k(table_hbm, idx_hbm, out_hbm, i0, i1, r0, r1, sg0, sg1, sw0, sw1):
        wid = lax.axis_index("s") * 2 + lax.axis_index("c")
        base = wid * per
        iv = (i0, i1)
        rv = (r0, r1)
        sg = (sg0, sg1)
        sw = (sw0, sw1)

        def wait_gather(b):
            pltpu.make_async_copy(table_hbm.at[pl.ds(0, _CH)], rv[b],
                                  sg[b]).wait()

        def wait_wb(b, off):
            pltpu.make_async_copy(rv[b], out_hbm.at[pl.ds(off, _CH)],
                                  sw[b]).wait()

        for b in range(2):
            pltpu.sync_copy(idx_hbm.at[pl.ds(base + b * _CH, _CH)], iv[b])
            pltpu.async_copy(table_hbm.at[iv[b]], rv[b], sg[b])

        def body(j2, carry):
            for b in range(2):
                off = base + (j2 * 2 + b) * _CH
                wait_gather(b)
                pltpu.async_copy(rv[b], out_hbm.at[pl.ds(off, _CH)], sw[b])
                pltpu.sync_copy(idx_hbm.at[pl.ds(off + 2 * _CH, _CH)], iv[b])
                wait_wb(b, off)
                pltpu.async_copy(table_hbm.at[iv[b]], rv[b], sg[b])
            return carry

        lax.fori_loop(0, nch // 2 - 1, body, 0)
        for b in range(2):
            off = base + (nch - 2 + b) * _CH
            wait_gather(b)
            pltpu.sync_copy(rv[b], out_hbm.at[pl.ds(off, _CH)])

    return k(table, idx)


_KB = 3
_KL = K1 // _KB


def _gstat_body(g_ref, valid_ref, tb_ref, st_ref, *, ct):
    b, kb = pl.program_id(0), pl.program_id(1)
    tb = tb_ref[0]
    s0 = jnp.zeros((1, ct), jnp.float32)
    s1 = jnp.zeros((1, ct), jnp.float32)
    vs = 0.0
    for kk in range(_KL):
        r = jnp.maximum(g_ref[0, kk][:, :ct] + tb, 0.0)
        v = valid_ref[0, kk, 0, :]
        rm = r * v[:, None]
        s0 = s0 + jnp.sum(rm, 0, keepdims=True)
        s1 = s1 + jnp.sum(rm * r, 0, keepdims=True)
        vs = vs + jnp.sum(v)

    @pl.when((b == 0) & (kb == 0))
    def _():
        st_ref[...] = jnp.zeros_like(st_ref)

    st_ref[0:1, :] = st_ref[0:1, :] + s0
    st_ref[1:2, :] = st_ref[1:2, :] + s1
    st_ref[2:3, :] = st_ref[2:3, :] + jnp.full((1, ct), vs)


def _gather_stats(g4, valid, tb):
    Cg = g4.shape[-1]
    ct = tb.shape[-1]
    return pl.pallas_call(
        functools.partial(_gstat_body, ct=ct),
        grid=(B, _KB),
        in_specs=[
            pl.BlockSpec((1, _KL, P, Cg), lambda b, k: (b, k, 0, 0)),
            pl.BlockSpec((1, _KL, 1, P), lambda b, k: (b, k, 0, 0)),
            pl.BlockSpec((1, P, ct), lambda b, k: (b, 0, 0)),
        ],
        out_specs=pl.BlockSpec((8, ct), lambda b, k: (0, 0)),
        out_shape=jax.ShapeDtypeStruct((8, ct), jnp.float32),
    )(g4, valid, tb)


def _midg_body(g_ref, valid_ref, tb_ref, sti_ref, w_ref, b_ref,
               r_ref, st_ref, *, ct):
    b, kb = pl.program_id(0), pl.program_id(1)
    tb = tb_ref[0]
    mu, inv = _finalize(sti_ref)
    w = w_ref[...]
    bv = b_ref[...]
    Cout = w.shape[1]
    s0 = jnp.zeros((1, Cout), jnp.float32)
    s1 = jnp.zeros((1, Cout), jnp.float32)
    vs = 0.0
    for kk in range(_KL):
        r1 = jnp.maximum(g_ref[0, kk][:, :ct] + tb, 0.0)
        h = (r1 - mu) * inv
        r = jnp.maximum(_dot(h, w) + bv, 0.0)
        r_ref[0, kk] = r
        v = valid_ref[0, kk, 0, :]
        rm = r * v[:, None]
        s0 = s0 + jnp.sum(rm, 0, keepdims=True)
        s1 = s1 + jnp.sum(rm * r, 0, keepdims=True)
        vs = vs + jnp.sum(v)

    @pl.when((b == 0) & (kb == 0))
    def _():
        st_ref[...] = jnp.zeros_like(st_ref)

    st_ref[0:1, :] = st_ref[0:1, :] + s0
    st_ref[1:2, :] = st_ref[1:2, :] + s1
    st_ref[2:3, :] = st_ref[2:3, :] + jnp.full((1, Cout), vs)


def _mid_gathered(g4, valid, tb, sti, w, bvec):
    Cg = g4.shape[-1]
    ct = tb.shape[-1]
    Cout = w.shape[1]
    return pl.pallas_call(
        functools.partial(_midg_body, ct=ct),
        grid=(B, _KB),
        in_specs=[
            pl.BlockSpec((1, _KL, P, Cg), lambda b, k: (b, k, 0, 0)),
            pl.BlockSpec((1, _KL, 1, P), lambda b, k: (b, k, 0, 0)),
            pl.BlockSpec((1, P, ct), lambda b, k: (b, 0, 0)),
            pl.BlockSpec((8, ct), lambda b, k: (0, 0)),
            pl.BlockSpec((ct, Cout), lambda b, k: (0, 0)),
            pl.BlockSpec((1, Cout), lambda b, k: (0, 0)),
        ],
        out_specs=[
            pl.BlockSpec((1, _KL, P, Cout), lambda b, k: (b, k, 0, 0)),
            pl.BlockSpec((8, Cout), lambda b, k: (0, 0)),
        ],
        out_shape=[
            jax.ShapeDtypeStruct((B, K1, P, Cout), jnp.float32),
            jax.ShapeDtypeStruct((8, Cout), jnp.float32),
        ],
    )(g4, valid, tb, sti, w, bvec)


def _mid_body(a_ref, v_ref, sti_ref, w_ref, b_ref, r_ref, st_ref):
    i = pl.program_id(0)
    mu, inv = _finalize(sti_ref)
    h = (a_ref[...] - mu) * inv
    z = _dot(h, w_ref[...]) + b_ref[...]
    r = jnp.maximum(z, 0.0)
    r_ref[...] = r
    v = v_ref[...]
    _accum_stats(st_ref, r, r * v, jnp.sum(v), i == 0)


def _mid(a, v, sti, w, bvec, bm):
    Mr, Cin = a.shape
    Cout = w.shape[1]
    return pl.pallas_call(
        _mid_body,
        grid=(Mr // bm,),
        in_specs=[
            pl.BlockSpec((bm, Cin), lambda i: (i, 0)),
            pl.BlockSpec((bm, 1), lambda i: (i, 0)),
            pl.BlockSpec((8, Cin), lambda i: (0, 0)),
            pl.BlockSpec((Cin, Cout), lambda i: (0, 0)),
            pl.BlockSpec((1, Cout), lambda i: (0, 0)),
        ],
        out_specs=[
            pl.BlockSpec((bm, Cout), lambda i: (i, 0)),
            pl.BlockSpec((8, Cout), lambda i: (0, 0)),
        ],
        out_shape=[
            jax.ShapeDtypeStruct((Mr, Cout), jnp.float32),
            jax.ShapeDtypeStruct((8, Cout), jnp.float32),
        ],
    )(a, v, sti, w, bvec)


def _mid2_body(a_ref, b2_ref, v_ref, sti_ref, wa_ref, wb_ref, b_ref,
               r_ref, st_ref):
    i = pl.program_id(0)
    mu, inv = _finalize(sti_ref)
    h = (a_ref[...] - mu) * inv
    z = _dot(h, wa_ref[...]) + _dot(b2_ref[...], wb_ref[...]) + b_ref[...]
    r = jnp.maximum(z, 0.0)
    r_ref[...] = r
    v = v_ref[...]
    _accum_stats(st_ref, r, r * v, jnp.sum(v), i == 0)


def _mid2(a, b2, v, sti, wa, wb, bvec, bm):
    Mr, Cin = a.shape
    Cb = b2.shape[1]
    Cout = wa.shape[1]
    return pl.pallas_call(
        _mid2_body,
        grid=(Mr // bm,),
        in_specs=[
            pl.BlockSpec((bm, Cin), lambda i: (i, 0)),
            pl.BlockSpec((bm, Cb), lambda i: (i, 0)),
            pl.BlockSpec((bm, 1), lambda i: (i, 0)),
            pl.BlockSpec((8, Cin), lambda i: (0, 0)),
            pl.BlockSpec((Cin, Cout), lambda i: (0, 0)),
            pl.BlockSpec((Cb, Cout), lambda i: (0, 0)),
            pl.BlockSpec((1, Cout), lambda i: (0, 0)),
        ],
        out_specs=[
            pl.BlockSpec((bm, Cout), lambda i: (i, 0)),
            pl.BlockSpec((8, Cout), lambda i: (0, 0)),
        ],
        out_shape=[
            jax.ShapeDtypeStruct((Mr, Cout), jnp.float32),
            jax.ShapeDtypeStruct((8, Cout), jnp.float32),
        ],
    )(a, b2, v, sti, wa, wb, bvec)


def _last_body(a_ref, valid_ref, sti_ref, w_ref, b_ref, pool_ref, st_ref):
    b, kb = pl.program_id(0), pl.program_id(1)
    mu, inv = _finalize(sti_ref)
    w = w_ref[...]
    bv = b_ref[...]
    Cout = w.shape[1]
    s0 = jnp.zeros((1, Cout), jnp.float32)
    s1 = jnp.zeros((1, Cout), jnp.float32)
    vs = 0.0

    @pl.when(kb == 0)
    def _():
        pool_ref[0] = jnp.full_like(pool_ref[0], -jnp.inf)

    acc = pool_ref[0]
    for kk in range(_KL):
        h = (a_ref[0, kk] - mu) * inv
        r = jnp.maximum(_dot(h, w) + bv, 0.0)
        v = valid_ref[0, kk, 0, :]
        rm = r * v[:, None]
        s0 = s0 + jnp.sum(rm, 0, keepdims=True)
        s1 = s1 + jnp.sum(rm * r, 0, keepdims=True)
        vs = vs + jnp.sum(v)
        acc = jnp.maximum(acc, jnp.where(v[:, None] > 0, r, -jnp.inf))
    pool_ref[0] = acc

    @pl.when((b == 0) & (kb == 0))
    def _():
        st_ref[...] = jnp.zeros_like(st_ref)

    st_ref[0:1, :] = st_ref[0:1, :] + s0
    st_ref[1:2, :] = st_ref[1:2, :] + s1
    st_ref[2:3, :] = st_ref[2:3, :] + jnp.full((1, Cout), vs)


def _conv_last(a4, valid, sti, w, bvec):
    Cin = a4.shape[-1]
    Cout = w.shape[1]
    return pl.pallas_call(
        _last_body,
        grid=(B, _KB),
        in_specs=[
            pl.BlockSpec((1, _KL, P, Cin), lambda b, k: (b, k, 0, 0)),
            pl.BlockSpec((1, _KL, 1, P), lambda b, k: (b, k, 0, 0)),
            pl.BlockSpec((8, Cin), lambda b, k: (0, 0)),
            pl.BlockSpec((Cin, Cout), lambda b, k: (0, 0)),
            pl.BlockSpec((1, Cout), lambda b, k: (0, 0)),
        ],
        out_specs=[
            pl.BlockSpec((1, P, Cout), lambda b, k: (b, 0, 0)),
            pl.BlockSpec((8, Cout), lambda b, k: (0, 0)),
        ],
        out_shape=[
            jax.ShapeDtypeStruct((B, P, Cout), jnp.float32),
            jax.ShapeDtypeStruct((8, Cout), jnp.float32),
        ],
    )(a4, valid, sti, w, bvec)


def _p4_body(a_ref, sti_ref, pp_ref, wx_ref, wp_ref, b_ref, s2_ref, tb2_ref):
    mu, inv = _finalize(sti_ref)
    h = (a_ref[...] - mu) * inv
    ppw = _dot(pp_ref[...], wp_ref[...])
    s2_ref[...] = _dot(h, wx_ref[...]) + ppw
    tb2_ref[...] = b_ref[...] - ppw


def _point_pre(a, sti, pp, wx, wp, bvec, bm):
    Mr, Cin = a.shape
    Cout = wx.shape[1]
    return pl.pallas_call(
        _p4_body,
        grid=(Mr // bm,),
        in_specs=[
            pl.BlockSpec((bm, Cin), lambda i: (i, 0)),
            pl.BlockSpec((8, Cin), lambda i: (0, 0)),
            pl.BlockSpec((bm, 8), lambda i: (i, 0)),
            pl.BlockSpec((Cin, Cout), lambda i: (0, 0)),
            pl.BlockSpec((8, Cout), lambda i: (0, 0)),
            pl.BlockSpec((1, Cout), lambda i: (0, 0)),
        ],
        out_specs=[
            pl.BlockSpec((bm, Cout), lambda i: (i, 0)),
            pl.BlockSpec((bm, Cout), lambda i: (i, 0)),
        ],
        out_shape=[
            jax.ShapeDtypeStruct((Mr, Cout), jnp.float32),
            jax.ShapeDtypeStruct((Mr, Cout), jnp.float32),
        ],
    )(a, sti, pp, wx, wp, bvec)


def _plast_body(a_ref, sti_ref, w_ref, b_ref, g_ref, st_ref):
    b = pl.program_id(0)
    mu, inv = _finalize(sti_ref)
    h = (a_ref[0] - mu) * inv
    z = _dot(h, w_ref[...]) + b_ref[...]
    r = jnp.maximum(z, 0.0)
    _accum_stats(st_ref, r, r, float(P), b == 0)
    g_ref[0, 0, :] = jnp.max(r, axis=0)


def _pool_last(a3, sti, w, bvec):
    Cin = a3.shape[-1]
    Cout = w.shape[1]
    return pl.pallas_call(
        _plast_body,
        grid=(B,),
        in_specs=[
            pl.BlockSpec((1, P, Cin), lambda b: (b, 0, 0)),
            pl.BlockSpec((8, Cin), lambda b: (0, 0)),
            pl.BlockSpec((Cin, Cout), lambda b: (0, 0)),
            pl.BlockSpec((1, Cout), lambda b: (0, 0)),
        ],
        out_specs=[
            pl.BlockSpec((1, 1, Cout), lambda b: (b, 0, 0)),
            pl.BlockSpec((8, Cout), lambda b: (0, 0)),
        ],
        out_shape=[
            jax.ShapeDtypeStruct((B, 1, Cout), jnp.float32),
            jax.ShapeDtypeStruct((8, Cout), jnp.float32),
        ],
    )(a3, sti, w, bvec)


def _head_body(g_ref, stp_ref, w1_ref, b1_ref, w2_ref, b2_ref, w3_ref,
               b3_ref, out_ref):
    mu, inv = _finalize(stp_ref)
    g = (g_ref[...] - mu) * inv
    h = jnp.maximum(_dot(g, w1_ref[...]) + b1_ref[...], 0.0)
    m1 = jnp.mean(h, 0, keepdims=True)
    v1 = jnp.mean((h - m1) ** 2, 0, keepdims=True)
    h = (h - m1) * lax.rsqrt(v1 + EPS)
    h = jnp.maximum(_dot(h, w2_ref[...]) + b2_ref[...], 0.0)
    m2 = jnp.mean(h, 0, keepdims=True)
    v2 = jnp.mean((h - m2) ** 2, 0, keepdims=True)
    h = (h - m2) * lax.rsqrt(v2 + EPS)
    out_ref[...] = _dot(h, w3_ref[...]) + b3_ref[...]


def _head(g, stp, w1, b1, w2, b2, w3p, b3p):
    return pl.pallas_call(
        _head_body,
        out_shape=jax.ShapeDtypeStruct((B, 128), jnp.float32),
    )(g, stp, w1, b1, w2, b2, w3p, b3p)


def _padrows(w, rows):
    return jnp.concatenate(
        [w, jnp.zeros((rows - w.shape[0], w.shape[1]), jnp.float32)], 0)


def kernel(x, pos, batch, params):
    f32 = jnp.float32
    x = x.astype(f32)
    pos = pos.astype(f32)
    xp = jnp.concatenate([x, pos, jnp.zeros((N, 2), f32)], 1).reshape(B, P, 8)
    posp = jnp.concatenate([pos, jnp.zeros((N, 5), f32)], 1).reshape(B, P, 8)
    post = jnp.swapaxes(posp, 1, 2)

    c1 = params["conv1"]
    w1p = _padrows(c1[0]["W"], 8)
    wp1 = _padrows(c1[0]["W"][3:6], 8)
    b1 = c1[0]["b"][None]

    nbrg, valid, s1, tb1 = _build_graph(posp, post, xp, w1p, wp1, b1)
    bmN = 2048 if N % 2048 == 0 else P

    idx = nbrg.reshape(M)
    s1p = jnp.concatenate([s1.reshape(N, 64), jnp.zeros((N, 64), f32)], 1)
    g1 = _sc_gather(s1p, idx).reshape(B, K1, P, 128)
    st1 = _gather_stats(g1, valid, tb1)
    r2, st2 = _mid_gathered(g1, valid, tb1, st1, c1[1]["W"], c1[1]["b"][None])
    pooled1, st3 = _conv_last(r2, valid, st2, c1[2]["W"], c1[2]["b"][None])

    c2 = params["conv2"]
    wx2 = c2[0]["W"][:128]
    wp2 = _padrows(c2[0]["W"][128:131], 8)
    s2, tb2 = _point_pre(pooled1.reshape(N, 128), st3, posp.reshape(N, 8),
                         wx2, wp2, c2[0]["b"][None], bm=bmN)

    tb2b = tb2.reshape(B, P, 128)
    g2 = _sc_gather(s2, idx).reshape(B, K1, P, 128)
    st1c = _gather_stats(g2, valid, tb2b)
    r2c, st2c = _mid_gathered(g2, valid, tb2b, st1c, c2[1]["W"],
                              c2[1]["b"][None])
    pooled2, st3c = _conv_last(r2c, valid, st2c, c2[2]["W"], c2[2]["b"][None])

    pw = params["pool"]
    wa = pw[0]["W"][:256]
    wb = _padrows(pw[0]["W"][256:259], 8)
    ones = jnp.ones((N, 1), f32)
    rp1, stp1 = _mid2(pooled2.reshape(N, 256), posp.reshape(N, 8), ones,
                      st3c, wa, wb, pw[0]["b"][None], bm=bmN)
    rp2, stp2 = _mid(rp1, ones, stp1, pw[1]["W"], pw[1]["b"][None], bm=bmN)
    g_pre, stp3 = _pool_last(rp2.reshape(B, P, 512), stp2, pw[2]["W"],
                             pw[2]["b"][None])

    fc1, fc2, fc3 = params["fc1"], params["fc2"], params["fc3"]
    w3p = jnp.concatenate([fc3["W"], jnp.zeros((256, 127), f32)], 1)
    b3p = jnp.concatenate([fc3["b"], jnp.zeros((127,), f32)])[None]
    out = _head(g_pre.reshape(B, 1024), stp3, fc1["W"], fc1["b"][None],
                fc2["W"], fc2["b"][None], w3p, b3p)
    return out[:, :1]
